# Initial kernel scaffold; baseline (speedup 1.0000x reference)
import jax
import jax.numpy as jnp
from jax.experimental import pallas as pl


def kernel(z, W):
    raise NotImplementedError("write your pallas kernel here")


# speedup vs baseline: 1.1698x; 1.1698x over previous
import functools

import jax
import jax.numpy as jnp
from jax import lax
from jax.experimental import pallas as pl
from jax.experimental.pallas import tpu as pltpu
from jax.experimental.pallas import tpu_sc as plsc

_N_EMBED = 8192
_EMBED_DIM = 32
_BETA = 0.25
_N_ROWS = 16384

_BLK_R = 512

_SC_NC = 2
_SC_NS = 16
_SC_NW = _SC_NC * _SC_NS
_B_PER_W = _N_ROWS // _SC_NW


_ARG_CHUNK = 4096


def _argmin_body(z_ref, w_ref, z2_ref, w2_ref, idx_ref, idxq_ref):
    zb = z_ref[...].astype(jnp.bfloat16)
    wb = w_ref[...].astype(jnp.bfloat16)
    m = lax.dot_general(
        zb, wb, (((1,), (1,)), ((), ())), preferred_element_type=jnp.float32
    )
    d = (z2_ref[...][:, None] + w2_ref[...][None, :]) - 2.0 * m
    run_val = jnp.full((z_ref.shape[0],), jnp.inf, jnp.float32)
    run_idx = jnp.zeros((z_ref.shape[0],), jnp.int32)
    for k in range(_N_EMBED // _ARG_CHUNK):
        blk = d[:, k * _ARG_CHUNK:(k + 1) * _ARG_CHUNK]
        cv = jnp.min(blk, axis=1)
        ii = lax.broadcasted_iota(jnp.int32, blk.shape, 1) + jnp.int32(
            k * _ARG_CHUNK
        )
        ci = jnp.min(
            jnp.where(blk == cv[:, None], ii, jnp.int32(_N_EMBED)), axis=1
        )
        new_wins = cv < run_val
        run_idx = jnp.where(new_wins, ci, run_idx)
        run_val = jnp.where(new_wins, cv, run_val)
        run_val = run_val.astype(jnp.bfloat16).astype(jnp.float32)
    idx_ref[...] = run_idx
    idxq_ref[...] = run_idx >> 2


def _sc_gather_body(table_hbm, idxq_hbm, out_hbm, idxq_v, rows_v, sem):
    wid = lax.axis_index("s") * _SC_NC + lax.axis_index("c")
    base = wid * _B_PER_W
    pltpu.sync_copy(idxq_hbm.at[pl.ds(base, _B_PER_W)], idxq_v)
    copies = []
    for j in range(_B_PER_W // 128):
        copies.append(
            pltpu.async_copy(
                table_hbm.at[idxq_v.at[pl.ds(j * 128, 128)]],
                rows_v.at[pl.ds(j * 128, 128)],
                sem,
            )
        )
    for c in copies:
        c.wait()
    pltpu.sync_copy(rows_v, out_hbm.at[pl.ds(base, _B_PER_W)])


def _finish_body(z_ref, zq4_ref, idx_ref, out_ref, loss_ref):
    zb = z_ref[...]
    rows = zq4_ref[...]
    r = (idx_ref[...] & 3)[:, None]
    zq = jnp.where(
        r == 0,
        rows[:, 0:32],
        jnp.where(
            r == 1,
            rows[:, 32:64],
            jnp.where(r == 2, rows[:, 64:96], rows[:, 96:128]),
        ),
    )
    diff = zq - zb
    out_ref[...] = zb + diff
    m = jnp.sum(diff * diff) / jnp.float32(_N_ROWS * _EMBED_DIM)
    loss_ref[0, 0] = m + jnp.float32(_BETA) * m


def kernel(z, W):
    z_flat = z.reshape(_N_ROWS, _EMBED_DIM)
    z2 = jnp.sum(z_flat**2, axis=1)
    w2 = jnp.sum(W**2, axis=1)

    indices, indices_q = pl.pallas_call(
        _argmin_body,
        grid=(_N_ROWS // _BLK_R,),
        in_specs=[
            pl.BlockSpec((_BLK_R, _EMBED_DIM), lambda i: (i, 0)),
            pl.BlockSpec((_N_EMBED, _EMBED_DIM), lambda i: (0, 0)),
            pl.BlockSpec((_BLK_R,), lambda i: (i,)),
            pl.BlockSpec((_N_EMBED,), lambda i: (0,)),
        ],
        out_specs=[
            pl.BlockSpec((_BLK_R,), lambda i: (i,)),
            pl.BlockSpec((_BLK_R,), lambda i: (i,)),
        ],
        out_shape=[
            jax.ShapeDtypeStruct((_N_ROWS,), jnp.int32),
            jax.ShapeDtypeStruct((_N_ROWS,), jnp.int32),
        ],
    )(z_flat, W, z2, w2)

    w4 = W.reshape(_N_EMBED // 4, 4 * _EMBED_DIM)
    z_q4 = pl.kernel(
        _sc_gather_body,
        mesh=plsc.VectorSubcoreMesh(core_axis_name="c", subcore_axis_name="s"),
        out_type=jax.ShapeDtypeStruct((_N_ROWS, 4 * _EMBED_DIM), jnp.float32),
        scratch_types=[
            pltpu.VMEM((_B_PER_W,), jnp.int32),
            pltpu.VMEM((_B_PER_W, 4 * _EMBED_DIM), jnp.float32),
            pltpu.SemaphoreType.DMA,
        ],
    )(w4, indices_q)

    z_q_st_flat, loss_arr = pl.pallas_call(
        _finish_body,
        in_specs=[
            pl.BlockSpec((_N_ROWS, _EMBED_DIM), lambda: (0, 0)),
            pl.BlockSpec((_N_ROWS, 4 * _EMBED_DIM), lambda: (0, 0)),
            pl.BlockSpec((_N_ROWS,), lambda: (0,)),
        ],
        out_specs=[
            pl.BlockSpec((_N_ROWS, _EMBED_DIM), lambda: (0, 0)),
            pl.BlockSpec((1, 1), lambda: (0, 0), memory_space=pltpu.SMEM),
        ],
        out_shape=[
            jax.ShapeDtypeStruct((_N_ROWS, _EMBED_DIM), jnp.float32),
            jax.ShapeDtypeStruct((1, 1), jnp.float32),
        ],
    )(z_flat, z_q4, indices)

    return (
        z_q_st_flat.reshape(z.shape),
        loss_arr[0, 0],
        indices.reshape(z.shape[:-1]),
    )
